# baseline (speedup 1.0000x reference)
import functools

import jax
import jax.numpy as jnp
from jax import lax
from jax.experimental import pallas as pl
from jax.experimental.pallas import tpu as pltpu
from jax.experimental.pallas import tpu_sc as plsc

NSUB = 128
NDIM = 3
NCOL = NSUB * NDIM
NROW = 150000
ROW_BLOCK = 2000
XPAD_ROWS = (NROW + 127) // 128 + 1
LANES = 16


def _sc_body(xpad_hbm, idx_hbm, xi_hbm, idx_v, tgt_v, rowidx_v, rows_v, xi_v,
             sem):
    cid = lax.axis_index("c")
    sid = lax.axis_index("s")

    @pl.when((cid == 0) & (sid == 0))
    def _():
        pltpu.sync_copy(idx_hbm, idx_v)
        for k in range(NSUB // LANES):
            ivec = idx_v[pl.ds(k * LANES, LANES)]
            base = ivec * 3
            lane = lax.iota(jnp.int32, LANES) + (k * LANES)
            for e in range(NDIM):
                tgt = base + e
                pos = lane * 3 + e
                plsc.store_scatter(tgt_v, [pos], tgt)
                plsc.store_scatter(
                    rowidx_v, [pos // 128, pos % 128], tgt // 128
                )
        for j in range(NCOL // 128):
            pltpu.async_copy(
                xpad_hbm.at[rowidx_v.at[j]], rows_v.at[j], sem
            ).wait()
        for k in range(NCOL // LANES):
            pos = lax.iota(jnp.int32, LANES) + k * LANES
            tgt = tgt_v[pl.ds(k * LANES, LANES)]
            word = plsc.load_gather(
                rows_v, [pos // 128, pos % 128, tgt % 128]
            )
            xi_v[pl.ds(k * LANES, LANES)] = word
        pltpu.sync_copy(xi_v, xi_hbm)


_sc_gather = functools.partial(
    pl.kernel,
    mesh=plsc.VectorSubcoreMesh(core_axis_name="c", subcore_axis_name="s"),
    out_type=jax.ShapeDtypeStruct((NCOL,), jnp.float32),
    scratch_types=[
        pltpu.VMEM((NSUB,), jnp.int32),
        pltpu.VMEM((NCOL,), jnp.int32),
        pltpu.VMEM((NCOL // 128, 128), jnp.int32),
        pltpu.VMEM((NCOL // 128, 128, 128), jnp.float32),
        pltpu.VMEM((NCOL,), jnp.float32),
        pltpu.SemaphoreType.DMA,
    ],
    compiler_params=pltpu.CompilerParams(
        needs_layout_passes=False, use_tc_tiling_on_sc=False
    ),
)(_sc_body)


def _grad_body(idx_ref, out_ref, tgt_ref):
    blk = pl.program_id(0)

    @pl.when(blk == 0)
    def _():
        ii = lax.broadcasted_iota(jnp.int32, (NSUB, NCOL), 0)
        cc = lax.broadcasted_iota(jnp.int32, (NSUB, NCOL), 1)
        onehot = (ii == cc // 3).astype(jnp.float32)
        idx_f = idx_ref[...].astype(jnp.float32)
        expanded = jnp.dot(
            idx_f,
            onehot,
            precision=lax.Precision.HIGHEST,
            preferred_element_type=jnp.float32,
        )
        col = lax.broadcasted_iota(jnp.int32, (1, NCOL), 1)
        tgt_ref[...] = expanded.astype(jnp.int32) * 3 + col % 3

    rows = lax.broadcasted_iota(jnp.int32, (ROW_BLOCK, NCOL), 0) + blk * ROW_BLOCK
    out_ref[...] = (rows == tgt_ref[...]).astype(jnp.float32)


_grad_call = pl.pallas_call(
    _grad_body,
    grid=(NROW // ROW_BLOCK,),
    in_specs=[pl.BlockSpec((1, NSUB), lambda i: (0, 0))],
    out_specs=pl.BlockSpec((ROW_BLOCK, NCOL), lambda i: (i, 0)),
    out_shape=jax.ShapeDtypeStruct((NROW, NCOL), jnp.float32),
    scratch_shapes=[pltpu.VMEM((1, NCOL), jnp.int32)],
)


def kernel(x, subset_indices):
    xpad = jnp.pad(x.reshape(-1), (0, XPAD_ROWS * 128 - NROW)).reshape(
        XPAD_ROWS, 128
    )
    xi = _sc_gather(xpad, subset_indices)
    grad = _grad_call(subset_indices.reshape(1, NSUB))
    return xi, grad


# speedup vs baseline: 1.0179x; 1.0179x over previous
import functools

import jax
import jax.numpy as jnp
from jax import lax
from jax.experimental import pallas as pl
from jax.experimental.pallas import tpu as pltpu
from jax.experimental.pallas import tpu_sc as plsc

NSUB = 128
NDIM = 3
NCOL = NSUB * NDIM
NROW = 150000
ROW_BLOCK = 5000
XPAD_ROWS = (NROW + 127) // 128 + 1
LANES = 16


def _sc_body(xpad_hbm, idx_hbm, xi_hbm, idx_v, tgt_v, rowidx_v, rows_v, xi_v,
             sem):
    cid = lax.axis_index("c")
    sid = lax.axis_index("s")

    @pl.when((cid == 0) & (sid == 0))
    def _():
        pltpu.sync_copy(idx_hbm, idx_v)
        for k in range(NSUB // LANES):
            ivec = idx_v[pl.ds(k * LANES, LANES)]
            base = ivec * 3
            lane = lax.iota(jnp.int32, LANES) + (k * LANES)
            for e in range(NDIM):
                tgt = base + e
                pos = lane * 3 + e
                plsc.store_scatter(tgt_v, [pos], tgt)
                plsc.store_scatter(
                    rowidx_v, [pos // 128, pos % 128], tgt // 128
                )
        for j in range(NCOL // 128):
            pltpu.async_copy(
                xpad_hbm.at[rowidx_v.at[j]], rows_v.at[j], sem
            ).wait()
        for k in range(NCOL // LANES):
            pos = lax.iota(jnp.int32, LANES) + k * LANES
            tgt = tgt_v[pl.ds(k * LANES, LANES)]
            word = plsc.load_gather(
                rows_v, [pos // 128, pos % 128, tgt % 128]
            )
            xi_v[pl.ds(k * LANES, LANES)] = word
        pltpu.sync_copy(xi_v, xi_hbm)


_sc_gather = functools.partial(
    pl.kernel,
    mesh=plsc.VectorSubcoreMesh(core_axis_name="c", subcore_axis_name="s"),
    out_type=jax.ShapeDtypeStruct((NCOL,), jnp.float32),
    scratch_types=[
        pltpu.VMEM((NSUB,), jnp.int32),
        pltpu.VMEM((NCOL,), jnp.int32),
        pltpu.VMEM((NCOL // 128, 128), jnp.int32),
        pltpu.VMEM((NCOL // 128, 128, 128), jnp.float32),
        pltpu.VMEM((NCOL,), jnp.float32),
        pltpu.SemaphoreType.DMA,
    ],
    compiler_params=pltpu.CompilerParams(
        needs_layout_passes=False, use_tc_tiling_on_sc=False
    ),
)(_sc_body)


def _grad_body(idx_ref, out_ref, tgt_ref):
    blk = pl.program_id(0)

    @pl.when(blk == 0)
    def _():
        ii = lax.broadcasted_iota(jnp.int32, (NSUB, NCOL), 0)
        cc = lax.broadcasted_iota(jnp.int32, (NSUB, NCOL), 1)
        onehot = (ii == cc // 3).astype(jnp.float32)
        idx_f = idx_ref[...].astype(jnp.float32)
        expanded = jnp.dot(
            idx_f,
            onehot,
            precision=lax.Precision.HIGHEST,
            preferred_element_type=jnp.float32,
        )
        col = lax.broadcasted_iota(jnp.int32, (1, NCOL), 1)
        tgt_ref[...] = expanded.astype(jnp.int32) * 3 + col % 3

    rows = lax.broadcasted_iota(jnp.int32, (ROW_BLOCK, NCOL), 0) + blk * ROW_BLOCK
    out_ref[...] = (rows == tgt_ref[...]).astype(jnp.float32)


_grad_call = pl.pallas_call(
    _grad_body,
    grid=(NROW // ROW_BLOCK,),
    in_specs=[pl.BlockSpec((1, NSUB), lambda i: (0, 0))],
    out_specs=pl.BlockSpec((ROW_BLOCK, NCOL), lambda i: (i, 0)),
    out_shape=jax.ShapeDtypeStruct((NROW, NCOL), jnp.float32),
    scratch_shapes=[pltpu.VMEM((1, NCOL), jnp.int32)],
)


def kernel(x, subset_indices):
    xpad = jnp.pad(x.reshape(-1), (0, XPAD_ROWS * 128 - NROW)).reshape(
        XPAD_ROWS, 128
    )
    xi = _sc_gather(xpad, subset_indices)
    grad = _grad_call(subset_indices.reshape(1, NSUB))
    return xi, grad


# speedup vs baseline: 1.0236x; 1.0056x over previous
import functools

import jax
import jax.numpy as jnp
from jax import lax
from jax.experimental import pallas as pl
from jax.experimental.pallas import tpu as pltpu
from jax.experimental.pallas import tpu_sc as plsc

NSUB = 128
NDIM = 3
NCOL = NSUB * NDIM
NROW = 150000
ROW_BLOCK = 3000
XPAD_ROWS = (NROW + 127) // 128 + 1
LANES = 16


def _sc_body(xpad_hbm, idx_hbm, xi_hbm, idx_v, tgt_v, rowidx_v, rows_v, xi_v,
             sem):
    cid = lax.axis_index("c")
    sid = lax.axis_index("s")

    @pl.when((cid == 0) & (sid == 0))
    def _():
        pltpu.sync_copy(idx_hbm, idx_v)
        for k in range(NSUB // LANES):
            ivec = idx_v[pl.ds(k * LANES, LANES)]
            base = ivec * 3
            lane = lax.iota(jnp.int32, LANES) + (k * LANES)
            for e in range(NDIM):
                tgt = base + e
                pos = lane * 3 + e
                plsc.store_scatter(tgt_v, [pos], tgt)
                plsc.store_scatter(
                    rowidx_v, [pos // 128, pos % 128], tgt // 128
                )
        for j in range(NCOL // 128):
            pltpu.async_copy(
                xpad_hbm.at[rowidx_v.at[j]], rows_v.at[j], sem
            ).wait()
        for k in range(NCOL // LANES):
            pos = lax.iota(jnp.int32, LANES) + k * LANES
            tgt = tgt_v[pl.ds(k * LANES, LANES)]
            word = plsc.load_gather(
                rows_v, [pos // 128, pos % 128, tgt % 128]
            )
            xi_v[pl.ds(k * LANES, LANES)] = word
        pltpu.sync_copy(xi_v, xi_hbm)


_sc_gather = functools.partial(
    pl.kernel,
    mesh=plsc.VectorSubcoreMesh(core_axis_name="c", subcore_axis_name="s"),
    out_type=jax.ShapeDtypeStruct((NCOL,), jnp.float32),
    scratch_types=[
        pltpu.VMEM((NSUB,), jnp.int32),
        pltpu.VMEM((NCOL,), jnp.int32),
        pltpu.VMEM((NCOL // 128, 128), jnp.int32),
        pltpu.VMEM((NCOL // 128, 128, 128), jnp.float32),
        pltpu.VMEM((NCOL,), jnp.float32),
        pltpu.SemaphoreType.DMA,
    ],
    compiler_params=pltpu.CompilerParams(
        needs_layout_passes=False, use_tc_tiling_on_sc=False
    ),
)(_sc_body)


def _grad_body(idx_ref, out_ref, tgt_ref):
    blk = pl.program_id(0)

    @pl.when(blk == 0)
    def _():
        ii = lax.broadcasted_iota(jnp.int32, (NSUB, NCOL), 0)
        cc = lax.broadcasted_iota(jnp.int32, (NSUB, NCOL), 1)
        onehot = (ii == cc // 3).astype(jnp.float32)
        idx_f = idx_ref[...].astype(jnp.float32)
        expanded = jnp.dot(
            idx_f,
            onehot,
            precision=lax.Precision.HIGHEST,
            preferred_element_type=jnp.float32,
        )
        col = lax.broadcasted_iota(jnp.int32, (1, NCOL), 1)
        tgt_ref[...] = expanded.astype(jnp.int32) * 3 + col % 3

    rows = lax.broadcasted_iota(jnp.int32, (ROW_BLOCK, NCOL), 0) + blk * ROW_BLOCK
    out_ref[...] = (rows == tgt_ref[...]).astype(jnp.float32)


_grad_call = pl.pallas_call(
    _grad_body,
    grid=(NROW // ROW_BLOCK,),
    in_specs=[pl.BlockSpec((1, NSUB), lambda i: (0, 0))],
    out_specs=pl.BlockSpec((ROW_BLOCK, NCOL), lambda i: (i, 0)),
    out_shape=jax.ShapeDtypeStruct((NROW, NCOL), jnp.float32),
    scratch_shapes=[pltpu.VMEM((1, NCOL), jnp.int32)],
)


def kernel(x, subset_indices):
    xpad = jnp.pad(x.reshape(-1), (0, XPAD_ROWS * 128 - NROW)).reshape(
        XPAD_ROWS, 128
    )
    xi = _sc_gather(xpad, subset_indices)
    grad = _grad_call(subset_indices.reshape(1, NSUB))
    return xi, grad
